# baseline (speedup 1.0000x reference)
import functools

import jax
import jax.numpy as jnp
from jax import lax
from jax.experimental import pallas as pl
from jax.experimental.pallas import tpu as pltpu
from jax.experimental.pallas import tpu_sc as plsc

NC = 2
NS = 16
NW = NC * NS
CHUNK = 128
LAYERS = 3
F0 = 0.42


def _make_spmm(n, d, epw_chunks, epw):
    mesh = plsc.VectorSubcoreMesh(core_axis_name="c", subcore_axis_name="s")
    rows_per_tile = n // NS

    @functools.partial(
        pl.kernel,
        mesh=mesh,
        compiler_params=pltpu.CompilerParams(needs_layout_passes=False),
        out_type=jax.ShapeDtypeStruct((NC, n, d), jnp.float32),
        scratch_types=[
            pltpu.VMEM((epw,), jnp.int32),
            pltpu.VMEM((epw,), jnp.float32),
            pltpu.VMEM((1, CHUNK), jnp.int32),
            pltpu.VMEM((1, CHUNK), jnp.int32),
            pltpu.VMEM((CHUNK, d), jnp.float32),
            pltpu.SemaphoreType.DMA,
            pltpu.SemaphoreType.DMA,
            pltpu.SemaphoreType.DMA,
            pltpu.VMEM_SHARED((n, d), jnp.float32),
        ],
    )
    def spmm(rows_hbm, cols_hbm, vals_hbm, x_hbm, zeros_hbm, p_hbm,
             cidx_v, w_v, rb_a, rb_b, msg_v, rsem_a, rsem_b, gsem, acc_sh):
        c = lax.axis_index("c")
        s = lax.axis_index("s")
        wid = s * NC + c

        pltpu.sync_copy(cols_hbm.at[wid], cidx_v)
        pltpu.sync_copy(vals_hbm.at[wid], w_v)

        pltpu.sync_copy(
            zeros_hbm.at[pl.ds(s * rows_per_tile, rows_per_tile)],
            acc_sh.at[pl.ds(s * rows_per_tile, rows_per_tile)],
        )
        plsc.subcore_barrier()

        rbs = (rb_a, rb_b)
        rsems = (rsem_a, rsem_b)

        def r_start(i, j):
            pltpu.async_copy(rows_hbm.at[wid, i], rbs[j], rsems[j])

        def r_wait(i, j):
            pltpu.make_async_copy(rows_hbm.at[wid, i], rbs[j], rsems[j]).wait()

        def do_chunk(i, j):
            cslice = cidx_v.at[pl.ds(i * CHUNK, CHUNK)]
            pltpu.async_copy(x_hbm.at[cslice], msg_v, gsem)
            pltpu.make_async_copy(x_hbm.at[cslice], msg_v, gsem).wait()

            def edge_body(e, carry2):
                widx = jnp.full((16,), i * CHUNK + e, jnp.int32)
                wvec = plsc.load_gather(w_v, [widx])
                for q in range(d // 16):
                    sl = pl.ds(q * 16, 16)
                    msg_v[e, sl] = msg_v[e, sl] * wvec
                return carry2

            lax.fori_loop(0, CHUNK, edge_body, 0, unroll=2)
            r_wait(i, j)
            pltpu.sync_copy(msg_v, acc_sh.at[rbs[j].at[0]], add=True)

        r_start(0, 0)
        r_start(1, 1)

        def group_body(t, carry):
            for j in (0, 1):
                i = 2 * t + j
                do_chunk(i, j)
                r_start(i + 2, j)
            return carry

        lax.fori_loop(0, epw_chunks // 2 - 1, group_body, 0)
        do_chunk(epw_chunks - 2, 0)
        do_chunk(epw_chunks - 1, 1)

        plsc.subcore_barrier()
        pltpu.sync_copy(
            acc_sh.at[pl.ds(s * rows_per_tile, rows_per_tile)],
            p_hbm.at[c, pl.ds(s * rows_per_tile, rows_per_tile)],
        )

    return spmm


def _make_combine(n, d, scale):
    blk = n // NS
    grid = (n // blk,)

    def body(p_ref, acc_ref, x_ref, accout_ref):
        x = p_ref[0] + p_ref[1]
        x_ref[...] = x
        accout_ref[...] = (acc_ref[...] + x) * scale

    return pl.pallas_call(
        body,
        grid=grid,
        in_specs=[
            pl.BlockSpec((2, blk, d), lambda i: (0, i, 0)),
            pl.BlockSpec((blk, d), lambda i: (i, 0)),
        ],
        out_specs=[
            pl.BlockSpec((blk, d), lambda i: (i, 0)),
            pl.BlockSpec((blk, d), lambda i: (i, 0)),
        ],
        out_shape=[
            jax.ShapeDtypeStruct((n, d), jnp.float32),
            jax.ShapeDtypeStruct((n, d), jnp.float32),
        ],
    )


def _layout(arr, epw, n0, n1):
    a0 = jnp.pad(arr[: NS * n0].reshape(NS, n0), ((0, 0), (0, epw - n0)))
    a1 = jnp.pad(arr[NS * n0:].reshape(NS, n1), ((0, 0), (0, epw - n1)))
    return jnp.stack([a0, a1], axis=1).reshape(NW, epw)


def kernel(edge_index, edge_values, embedding):
    e = edge_values.shape[0]
    n, d = embedding.shape

    n0 = int(e * F0) // NS
    n1 = (e - NS * n0 + NS - 1) // NS
    per = 2 * CHUNK
    epw = ((max(n0, n1) + per - 1) // per) * per
    pad_to = NS * (n0 + n1)
    rows_f = jnp.pad(edge_index[0], (0, pad_to - e))
    cols_f = jnp.pad(edge_index[1], (0, pad_to - e))
    vals_f = jnp.pad(edge_values, (0, pad_to - e))
    rows = _layout(rows_f, epw, n0, n1).reshape(NW, epw // CHUNK, 1, CHUNK)
    cols = _layout(cols_f, epw, n0, n1)
    vals = _layout(vals_f, epw, n0, n1)

    align = 8 * NS
    npad = ((n + align - 1) // align) * align
    x0 = jnp.pad(embedding, ((0, npad - n), (0, 0)))
    zeros = jnp.zeros((npad, d), jnp.float32)

    spmm = _make_spmm(npad, d, epw // CHUNK, epw)

    x = x0
    acc = x0
    for layer in range(LAYERS):
        p = spmm(rows, cols, vals, x, zeros)
        scale = 1.0 / (LAYERS + 1) if layer == LAYERS - 1 else 1.0
        x, acc = _make_combine(npad, d, scale)(p, acc)
    return acc[:n]


# speedup vs baseline: 5.5939x; 5.5939x over previous
import functools

import jax
import jax.numpy as jnp
from jax import lax
from jax.experimental import pallas as pl
from jax.experimental.pallas import tpu as pltpu
from jax.experimental.pallas import tpu_sc as plsc

NC = 2
NS = 16
NW = NC * NS
CHUNK = 128
NSPLIT = 2
LAYERS = 3


def _make_spmm(n, d, epw_chunks):
    mesh = plsc.VectorSubcoreMesh(core_axis_name="c", subcore_axis_name="s")
    rows_per_tile = n // NS
    part = CHUNK // NSPLIT

    @functools.partial(
        pl.kernel,
        mesh=mesh,
        compiler_params=pltpu.CompilerParams(needs_layout_passes=False),
        out_type=jax.ShapeDtypeStruct((NC, n, d), jnp.float32),
        scratch_types=[
            pltpu.VMEM((epw_chunks, CHUNK), jnp.int32),
            pltpu.VMEM((epw_chunks, CHUNK), jnp.int32),
            pltpu.VMEM((epw_chunks, CHUNK), jnp.float32),
            pltpu.VMEM((CHUNK, d), jnp.float32),
            pltpu.SemaphoreType.DMA,
            pltpu.SemaphoreType.DMA,
            pltpu.VMEM_SHARED((n, d), jnp.float32),
        ],
    )
    def spmm(rows_hbm, cols_hbm, vals_hbm, x_hbm, zeros_hbm, p_hbm,
             ridx_v, cidx_v, w_v, msg_v, gsem_a, gsem_b, acc_sh):
        c = lax.axis_index("c")
        s = lax.axis_index("s")
        wid = s * NC + c

        pltpu.sync_copy(rows_hbm.at[wid], ridx_v)
        pltpu.sync_copy(cols_hbm.at[wid], cidx_v)
        pltpu.sync_copy(vals_hbm.at[wid], w_v)

        pltpu.sync_copy(
            zeros_hbm.at[pl.ds(s * rows_per_tile, rows_per_tile)],
            acc_sh.at[pl.ds(s * rows_per_tile, rows_per_tile)],
        )
        plsc.subcore_barrier()

        gsems = (gsem_a, gsem_b)

        def chunk_body(i, carry):
            for k in range(NSPLIT):
                pltpu.async_copy(
                    x_hbm.at[cidx_v.at[i, pl.ds(k * part, part)]],
                    msg_v.at[pl.ds(k * part, part)],
                    gsems[k],
                )
            for k in range(NSPLIT):
                pltpu.make_async_copy(
                    x_hbm.at[cidx_v.at[i, pl.ds(k * part, part)]],
                    msg_v.at[pl.ds(k * part, part)],
                    gsems[k],
                ).wait()

            def edge_body(e, carry2):
                widx = jnp.full((16,), e, jnp.int32)
                wvec = plsc.load_gather(w_v.at[i], [widx])
                for j in range(d // 16):
                    sl = pl.ds(j * 16, 16)
                    msg_v[e, sl] = msg_v[e, sl] * wvec
                return carry2

            lax.fori_loop(0, CHUNK, edge_body, 0, unroll=2)
            pltpu.sync_copy(msg_v, acc_sh.at[ridx_v.at[i]], add=True)
            return carry

        lax.fori_loop(0, epw_chunks, chunk_body, 0)
        plsc.subcore_barrier()
        pltpu.sync_copy(
            acc_sh.at[pl.ds(s * rows_per_tile, rows_per_tile)],
            p_hbm.at[c, pl.ds(s * rows_per_tile, rows_per_tile)],
        )

    return spmm


def _make_combine(n, d, scale):
    blk = n // NS
    grid = (n // blk,)

    def body(p_ref, acc_ref, x_ref, accout_ref):
        x = p_ref[0] + p_ref[1]
        x_ref[...] = x
        accout_ref[...] = (acc_ref[...] + x) * scale

    return pl.pallas_call(
        body,
        grid=grid,
        in_specs=[
            pl.BlockSpec((2, blk, d), lambda i: (0, i, 0)),
            pl.BlockSpec((blk, d), lambda i: (i, 0)),
        ],
        out_specs=[
            pl.BlockSpec((blk, d), lambda i: (i, 0)),
            pl.BlockSpec((blk, d), lambda i: (i, 0)),
        ],
        out_shape=[
            jax.ShapeDtypeStruct((n, d), jnp.float32),
            jax.ShapeDtypeStruct((n, d), jnp.float32),
        ],
    )


def kernel(edge_index, edge_values, embedding):
    e = edge_values.shape[0]
    n, d = embedding.shape

    per = NW * CHUNK
    epad = ((e + per - 1) // per) * per
    pad = epad - e
    epw = epad // NW
    rows = jnp.pad(edge_index[0], (0, pad)).reshape(NW, epw // CHUNK, CHUNK)
    cols = jnp.pad(edge_index[1], (0, pad)).reshape(NW, epw // CHUNK, CHUNK)
    vals = jnp.pad(edge_values, (0, pad)).reshape(NW, epw // CHUNK, CHUNK)

    align = 8 * NS
    npad = ((n + align - 1) // align) * align
    x0 = jnp.pad(embedding, ((0, npad - n), (0, 0)))
    zeros = jnp.zeros((npad, d), jnp.float32)

    epw_chunks = epw // CHUNK
    spmm = _make_spmm(npad, d, epw_chunks)

    x = x0
    acc = x0
    for layer in range(LAYERS):
        p = spmm(rows, cols, vals, x, zeros)
        scale = 1.0 / (LAYERS + 1) if layer == LAYERS - 1 else 1.0
        x, acc = _make_combine(npad, d, scale)(p, acc)
    return acc[:n]
